# Initial kernel scaffold; baseline (speedup 1.0000x reference)
import jax
import jax.numpy as jnp
from jax.experimental import pallas as pl


def kernel(x, gate_w, Wg, Wu, Wd):
    raise NotImplementedError("write your pallas kernel here")


# speedup vs baseline: 2.5180x; 2.5180x over previous
import functools

import jax
import jax.numpy as jnp
from jax.experimental import pallas as pl
from jax.experimental.pallas import tpu as pltpu

NUM_EXPERTS = 64
TOP_K = 2
D_MODEL = 768
D_FF = 1024
SEQ = 2048
BM = 128
N_ASSIGN = SEQ * TOP_K
NUM_BLOCKS = N_ASSIGN // BM + NUM_EXPERTS


def _moe_block_kernel(be_ref, act_ref, xs_ref, wg_ref, wu_ref, wd_ref, out_ref):
    i = pl.program_id(0)

    @pl.when(act_ref[i] == 1)
    def _():
        xb = xs_ref[...]
        g = jnp.dot(xb, wg_ref[0], preferred_element_type=jnp.float32)
        u = jnp.dot(xb, wu_ref[0], preferred_element_type=jnp.float32)
        h = (g * jax.nn.sigmoid(g)) * u
        out_ref[...] = jnp.dot(h, wd_ref[0], preferred_element_type=jnp.float32)


@jax.jit
def kernel(x, gate_w, Wg, Wu, Wd):
    x0 = x[0]

    scores = x0 @ gate_w
    topv, topi = jax.lax.top_k(scores, TOP_K)
    tw = jax.nn.softmax(topv, axis=-1)

    e_flat = topi.reshape(-1).astype(jnp.int32)
    t_flat = jnp.repeat(jnp.arange(SEQ, dtype=jnp.int32), TOP_K)
    w_flat = tw.reshape(-1)

    order = jnp.argsort(e_flat)
    se = e_flat[order]
    st = t_flat[order]
    sw = w_flat[order]

    gs = jnp.zeros((NUM_EXPERTS,), jnp.int32).at[se].add(1)
    blocks_per = (gs + BM - 1) // BM
    bcum = jnp.cumsum(blocks_per)
    bstart = bcum - blocks_per
    n_active = bcum[-1]

    goff = jnp.cumsum(gs) - gs
    rank = jnp.arange(N_ASSIGN, dtype=jnp.int32) - goff[se]
    p = bstart[se] * BM + rank

    tok_pad = jnp.full((NUM_BLOCKS * BM,), SEQ, jnp.int32).at[p].set(st)
    w_pad = jnp.zeros((NUM_BLOCKS * BM,), jnp.float32).at[p].set(sw)

    blk_ids = jnp.arange(NUM_BLOCKS, dtype=jnp.int32)
    be = jnp.searchsorted(bcum, blk_ids, side="right").astype(jnp.int32)
    last_e = se[N_ASSIGN - 1]
    block_expert = jnp.where(
        blk_ids < n_active, jnp.minimum(be, NUM_EXPERTS - 1), last_e
    ).astype(jnp.int32)
    active = (blk_ids < n_active).astype(jnp.int32)

    xs_pad = jnp.take(x0, jnp.minimum(tok_pad, SEQ - 1), axis=0)

    grid_spec = pltpu.PrefetchScalarGridSpec(
        num_scalar_prefetch=2,
        grid=(NUM_BLOCKS,),
        in_specs=[
            pl.BlockSpec((BM, D_MODEL), lambda i, be_r, a_r: (i, 0)),
            pl.BlockSpec((1, D_MODEL, D_FF), lambda i, be_r, a_r: (be_r[i], 0, 0)),
            pl.BlockSpec((1, D_MODEL, D_FF), lambda i, be_r, a_r: (be_r[i], 0, 0)),
            pl.BlockSpec((1, D_FF, D_MODEL), lambda i, be_r, a_r: (be_r[i], 0, 0)),
        ],
        out_specs=pl.BlockSpec((BM, D_MODEL), lambda i, be_r, a_r: (i, 0)),
    )

    out_pad = pl.pallas_call(
        _moe_block_kernel,
        grid_spec=grid_spec,
        out_shape=jax.ShapeDtypeStruct((NUM_BLOCKS * BM, D_MODEL), jnp.float32),
    )(block_expert, active, xs_pad, Wg, Wu, Wd)

    out = (
        jnp.zeros((SEQ, D_MODEL), jnp.float32)
        .at[tok_pad]
        .add(out_pad * w_pad[:, None], mode="drop")
    )
    return out[None]


# speedup vs baseline: 3.5939x; 1.4273x over previous
import functools

import jax
import jax.numpy as jnp
from jax.experimental import pallas as pl
from jax.experimental.pallas import tpu as pltpu

NUM_EXPERTS = 64
TOP_K = 2
D_MODEL = 768
D_FF = 1024
SEQ = 2048
BM = 128
N_ASSIGN = SEQ * TOP_K
NUM_BLOCKS = N_ASSIGN // BM + NUM_EXPERTS


def _moe_block_kernel(be_ref, act_ref, xs_ref, wg_ref, wu_ref, wd_ref, out_ref):
    i = pl.program_id(0)

    @pl.when(act_ref[i] == 1)
    def _():
        xb = xs_ref[...]
        g = jnp.dot(xb, wg_ref[0], preferred_element_type=jnp.float32)
        u = jnp.dot(xb, wu_ref[0], preferred_element_type=jnp.float32)
        h = (g * jax.nn.sigmoid(g)) * u
        out_ref[...] = jnp.dot(h, wd_ref[0], preferred_element_type=jnp.float32)


@jax.jit
def kernel(x, gate_w, Wg, Wu, Wd):
    x0 = x[0]

    scores = x0 @ gate_w
    topv, topi = jax.lax.top_k(scores, TOP_K)
    tw = jax.nn.softmax(topv, axis=-1)

    e_flat = topi.reshape(-1).astype(jnp.int32)
    t_flat = jnp.repeat(jnp.arange(SEQ, dtype=jnp.int32), TOP_K)
    w_flat = tw.reshape(-1)

    order = jnp.argsort(e_flat)
    se = e_flat[order]
    st = t_flat[order]
    sw = w_flat[order]

    gs = jnp.zeros((NUM_EXPERTS,), jnp.int32).at[se].add(1)
    blocks_per = (gs + BM - 1) // BM
    bcum = jnp.cumsum(blocks_per)
    bstart = bcum - blocks_per
    n_active = bcum[-1]

    goff = jnp.cumsum(gs) - gs
    rank = jnp.arange(N_ASSIGN, dtype=jnp.int32) - goff[se]
    p = bstart[se] * BM + rank

    tok_pad = jnp.full((NUM_BLOCKS * BM,), SEQ, jnp.int32).at[p].set(st)
    inv_p = jnp.zeros((N_ASSIGN,), jnp.int32).at[order].set(p)
    pos = inv_p.reshape(SEQ, TOP_K)

    blk_ids = jnp.arange(NUM_BLOCKS, dtype=jnp.int32)
    be = jnp.searchsorted(bcum, blk_ids, side="right").astype(jnp.int32)
    last_e = se[N_ASSIGN - 1]
    block_expert = jnp.where(
        blk_ids < n_active, jnp.minimum(be, NUM_EXPERTS - 1), last_e
    ).astype(jnp.int32)
    active = (blk_ids < n_active).astype(jnp.int32)

    xs_pad = jnp.take(x0, jnp.minimum(tok_pad, SEQ - 1), axis=0)

    grid_spec = pltpu.PrefetchScalarGridSpec(
        num_scalar_prefetch=2,
        grid=(NUM_BLOCKS,),
        in_specs=[
            pl.BlockSpec((BM, D_MODEL), lambda i, be_r, a_r: (i, 0)),
            pl.BlockSpec((1, D_MODEL, D_FF), lambda i, be_r, a_r: (be_r[i], 0, 0)),
            pl.BlockSpec((1, D_MODEL, D_FF), lambda i, be_r, a_r: (be_r[i], 0, 0)),
            pl.BlockSpec((1, D_FF, D_MODEL), lambda i, be_r, a_r: (be_r[i], 0, 0)),
        ],
        out_specs=pl.BlockSpec((BM, D_MODEL), lambda i, be_r, a_r: (i, 0)),
    )

    out_pad = pl.pallas_call(
        _moe_block_kernel,
        grid_spec=grid_spec,
        out_shape=jax.ShapeDtypeStruct((NUM_BLOCKS * BM, D_MODEL), jnp.float32),
    )(block_expert, active, xs_pad, Wg, Wu, Wd)

    out = tw[:, 0, None] * out_pad[pos[:, 0]] + tw[:, 1, None] * out_pad[pos[:, 1]]
    return out[None]


# speedup vs baseline: 5.1674x; 1.4378x over previous
import functools

import jax
import jax.numpy as jnp
from jax.experimental import pallas as pl
from jax.experimental.pallas import tpu as pltpu

NUM_EXPERTS = 64
TOP_K = 2
D_MODEL = 768
D_FF = 1024
SEQ = 2048
BM = 128
N_ASSIGN = SEQ * TOP_K
NUM_BLOCKS = N_ASSIGN // BM + NUM_EXPERTS


def _moe_block_kernel(
    be_ref, act_ref, x_ref, tok_col_ref, tok_row_ref, w_row_ref,
    wg_ref, wu_ref, wd_ref, out_ref
):
    i = pl.program_id(0)

    @pl.when(i == 0)
    def _():
        out_ref[...] = jnp.zeros_like(out_ref)

    @pl.when(act_ref[i] == 1)
    def _():
        ids_col = tok_col_ref[...]
        gmat = (
            jax.lax.broadcasted_iota(jnp.int32, (BM, SEQ), 1) == ids_col
        ).astype(jnp.float32)
        xb = jnp.dot(gmat, x_ref[...], preferred_element_type=jnp.float32)

        g = jnp.dot(xb, wg_ref[0], preferred_element_type=jnp.float32)
        u = jnp.dot(xb, wu_ref[0], preferred_element_type=jnp.float32)
        h = (g * jax.nn.sigmoid(g)) * u
        y = jnp.dot(h, wd_ref[0], preferred_element_type=jnp.float32)

        ids_row = tok_row_ref[0, 0:1, :]
        w_row = w_row_ref[0, 0:1, :]
        smat = (
            jax.lax.broadcasted_iota(jnp.int32, (SEQ, BM), 0) == ids_row
        ).astype(jnp.float32) * w_row
        out_ref[...] += jnp.dot(smat, y, preferred_element_type=jnp.float32)


@jax.jit
def kernel(x, gate_w, Wg, Wu, Wd):
    x0 = x[0]

    scores = x0 @ gate_w
    eids = jnp.arange(NUM_EXPERTS, dtype=jnp.int32)
    i1 = jnp.argmax(scores, axis=-1).astype(jnp.int32)
    v1 = jnp.max(scores, axis=-1)
    masked = jnp.where(eids[None, :] == i1[:, None], -jnp.inf, scores)
    i2 = jnp.argmax(masked, axis=-1).astype(jnp.int32)
    v2 = jnp.max(masked, axis=-1)
    s1 = jax.nn.sigmoid(v1 - v2)
    tw = jnp.stack([s1, 1.0 - s1], axis=-1)

    e_flat = jnp.stack([i1, i2], axis=-1).reshape(-1)
    t_flat = jnp.repeat(jnp.arange(SEQ, dtype=jnp.int32), TOP_K)

    onehot = (e_flat[:, None] == eids[None, :]).astype(jnp.int32)
    csum = jnp.cumsum(onehot, axis=0)
    gs = csum[-1]
    rank = jnp.sum((csum - onehot) * onehot, axis=1)

    blocks_per = (gs + BM - 1) // BM
    bcum = jnp.cumsum(blocks_per)
    bstart = bcum - blocks_per
    n_active = bcum[-1]

    p = bstart[e_flat] * BM + rank
    tok_pad = jnp.full((NUM_BLOCKS * BM,), SEQ, jnp.int32).at[p].set(t_flat)
    w_pad = jnp.zeros((NUM_BLOCKS * BM,), jnp.float32).at[p].set(tw.reshape(-1))

    blk_ids = jnp.arange(NUM_BLOCKS, dtype=jnp.int32)
    be = jnp.searchsorted(
        bcum, jnp.minimum(blk_ids, n_active - 1), side="right"
    ).astype(jnp.int32)
    block_expert = jnp.minimum(be, NUM_EXPERTS - 1)
    active = (blk_ids < n_active).astype(jnp.int32)

    grid_spec = pltpu.PrefetchScalarGridSpec(
        num_scalar_prefetch=2,
        grid=(NUM_BLOCKS,),
        in_specs=[
            pl.BlockSpec((SEQ, D_MODEL), lambda i, be_r, a_r: (0, 0)),
            pl.BlockSpec((BM, 1), lambda i, be_r, a_r: (i, 0)),
            pl.BlockSpec((1, 8, BM), lambda i, be_r, a_r: (i, 0, 0)),
            pl.BlockSpec((1, 8, BM), lambda i, be_r, a_r: (i, 0, 0)),
            pl.BlockSpec((1, D_MODEL, D_FF), lambda i, be_r, a_r: (be_r[i], 0, 0)),
            pl.BlockSpec((1, D_MODEL, D_FF), lambda i, be_r, a_r: (be_r[i], 0, 0)),
            pl.BlockSpec((1, D_FF, D_MODEL), lambda i, be_r, a_r: (be_r[i], 0, 0)),
        ],
        out_specs=pl.BlockSpec((SEQ, D_MODEL), lambda i, be_r, a_r: (0, 0)),
    )

    out = pl.pallas_call(
        _moe_block_kernel,
        grid_spec=grid_spec,
        out_shape=jax.ShapeDtypeStruct((SEQ, D_MODEL), jnp.float32),
    )(
        block_expert,
        active,
        x0,
        tok_pad.reshape(NUM_BLOCKS * BM, 1),
        jnp.broadcast_to(tok_pad.reshape(NUM_BLOCKS, 1, BM), (NUM_BLOCKS, 8, BM)),
        jnp.broadcast_to(w_pad.reshape(NUM_BLOCKS, 1, BM), (NUM_BLOCKS, 8, BM)),
        Wg,
        Wu,
        Wd,
    )
    return out[None]
